# Initial kernel scaffold; baseline (speedup 1.0000x reference)
import jax
import jax.numpy as jnp
from jax.experimental import pallas as pl


def kernel(lut, img):
    raise NotImplementedError("write your pallas kernel here")


# speedup vs baseline: 743.2252x; 743.2252x over previous
import functools

import jax
import jax.numpy as jnp
from jax import lax
from jax.experimental import pallas as pl
from jax.experimental.pallas import tpu as pltpu
from jax.experimental.pallas import tpu_sc as plsc

H, W = 1080, 1920
NPIX = H * W
NLUT = 33
LUT_C = NLUT * NLUT * NLUT
LUT_WORDS = 3 * LUT_C
NC, NS, L = 2, 16, 16
NW = NC * NS
PIX_PER_TILE = NPIX // NW
P = 2160
NCHUNK = PIX_PER_TILE // P
VPC = P // L

_mesh = plsc.VectorSubcoreMesh(core_axis_name="c", subcore_axis_name="s")


@functools.partial(
    pl.kernel,
    mesh=_mesh,
    compiler_params=pltpu.CompilerParams(needs_layout_passes=False),
    out_type=jax.ShapeDtypeStruct((3 * NPIX,), jnp.float32),
    scratch_types=[
        pltpu.VMEM((LUT_WORDS,), jnp.float32),
        pltpu.VMEM((P,), jnp.float32),
        pltpu.VMEM((P,), jnp.float32),
        pltpu.VMEM((P,), jnp.float32),
        pltpu.VMEM((P,), jnp.float32),
        pltpu.VMEM((P,), jnp.float32),
        pltpu.VMEM((P,), jnp.float32),
    ],
)
def _interp(lut_hbm, img_hbm, out_hbm, lut_v, r_v, g_v, b_v, o0_v, o1_v, o2_v):
    wid = lax.axis_index("s") * NC + lax.axis_index("c")
    pltpu.sync_copy(lut_hbm, lut_v)
    tile_base = wid * PIX_PER_TILE

    def chunk_body(ci, carry):
        start = tile_base + ci * P
        pltpu.sync_copy(img_hbm.at[pl.ds(start, P)], r_v)
        pltpu.sync_copy(img_hbm.at[pl.ds(NPIX + start, P)], g_v)
        pltpu.sync_copy(img_hbm.at[pl.ds(2 * NPIX + start, P)], b_v)

        def vec_body(j, c2):
            o = j * L
            r = r_v[pl.ds(o, L)]
            g = g_v[pl.ds(o, L)]
            b = b_v[pl.ds(o, L)]
            fx = jnp.minimum(jnp.maximum(r * 32.0, 0.0), 32.0)
            fy = jnp.minimum(jnp.maximum(g * 32.0, 0.0), 32.0)
            fz = jnp.minimum(jnp.maximum(b * 32.0, 0.0), 32.0)
            ix0 = fx.astype(jnp.int32)
            iy0 = fy.astype(jnp.int32)
            iz0 = fz.astype(jnp.int32)
            wx = fx - ix0.astype(jnp.float32)
            wy = fy - iy0.astype(jnp.float32)
            wz = fz - iz0.astype(jnp.float32)
            dx = jnp.minimum(ix0 + 1, NLUT - 1) - ix0
            dy = (jnp.minimum(iy0 + 1, NLUT - 1) - iy0) * NLUT
            dz = (jnp.minimum(iz0 + 1, NLUT - 1) - iz0) * (NLUT * NLUT)
            base = iz0 * (NLUT * NLUT) + iy0 * NLUT + ix0
            i001 = base + dx
            i010 = base + dy
            i011 = i010 + dx
            i100 = base + dz
            i101 = i100 + dx
            i110 = i100 + dy
            i111 = i110 + dx
            ux = 1.0 - wx
            uy = 1.0 - wy
            uz = 1.0 - wz
            wy0x0 = uy * ux
            wy0x1 = uy * wx
            wy1x0 = wy * ux
            wy1x1 = wy * wx
            w000 = uz * wy0x0
            w001 = uz * wy0x1
            w010 = uz * wy1x0
            w011 = uz * wy1x1
            w100 = wz * wy0x0
            w101 = wz * wy0x1
            w110 = wz * wy1x0
            w111 = wz * wy1x1
            for c, o_ref in ((0, o0_v), (1, o1_v), (2, o2_v)):
                off = c * LUT_C
                acc = plsc.load_gather(lut_v, [base + off]) * w000
                acc += plsc.load_gather(lut_v, [i001 + off]) * w001
                acc += plsc.load_gather(lut_v, [i010 + off]) * w010
                acc += plsc.load_gather(lut_v, [i011 + off]) * w011
                acc += plsc.load_gather(lut_v, [i100 + off]) * w100
                acc += plsc.load_gather(lut_v, [i101 + off]) * w101
                acc += plsc.load_gather(lut_v, [i110 + off]) * w110
                acc += plsc.load_gather(lut_v, [i111 + off]) * w111
                o_ref[pl.ds(o, L)] = acc
            return c2

        lax.fori_loop(0, VPC, vec_body, 0)
        pltpu.sync_copy(o0_v, out_hbm.at[pl.ds(start, P)])
        pltpu.sync_copy(o1_v, out_hbm.at[pl.ds(NPIX + start, P)])
        pltpu.sync_copy(o2_v, out_hbm.at[pl.ds(2 * NPIX + start, P)])
        return carry

    lax.fori_loop(0, NCHUNK, chunk_body, 0)


def kernel(lut, img):
    lut_flat = lut.reshape(LUT_WORDS)
    img_flat = img.reshape(3 * NPIX)
    out = _interp(lut_flat, img_flat)
    return (lut[None], out.reshape(1, 3, H, W))
